# Initial kernel scaffold; baseline (speedup 1.0000x reference)
import jax
import jax.numpy as jnp
from jax.experimental import pallas as pl


def kernel(x, src, neg_dst, labels, fc_w, attn_l, attn_r):
    raise NotImplementedError("write your pallas kernel here")


# speedup vs baseline: 3.5507x; 3.5507x over previous
import dataclasses
import functools

import jax
import jax.numpy as jnp
from jax import lax
from jax.experimental import pallas as pl
from jax.experimental.pallas import tpu as pltpu
from jax.experimental.pallas import tpu_sc as plsc

N, DEG, DIN, DOUT, H, K = 10000, 16, 128, 128, 4, 8
GM, CM, SLOPE = 0.1, 0.1, 0.2
E = N * DEG
EP = 1280 * 128
NW = 32

BA = 400
BC = 40


def _stage_a_body(x_ref, w_ref, al_ref, ar_ref, el_ref, er_ref):
    xb = x_ref[...]
    hb = jnp.dot(xb, w_ref[...], preferred_element_type=jnp.float32)
    h4 = hb.reshape(BA, H, DOUT)
    el_ref[...] = (h4 * al_ref[...][None]).sum(-1)
    er_ref[...] = (h4 * ar_ref[...][None]).sum(-1)


def _stage_a(x, fcwT, al, ar):
    return pl.pallas_call(
        _stage_a_body,
        grid=(N // BA,),
        in_specs=[
            pl.BlockSpec((BA, DIN), lambda i: (i, 0)),
            pl.BlockSpec((DIN, H * DOUT), lambda i: (0, 0)),
            pl.BlockSpec((H, DOUT), lambda i: (0, 0)),
            pl.BlockSpec((H, DOUT), lambda i: (0, 0)),
        ],
        out_specs=[
            pl.BlockSpec((BA, H), lambda i: (i, 0)),
            pl.BlockSpec((BA, H), lambda i: (i, 0)),
        ],
        out_shape=[
            jax.ShapeDtypeStruct((N, H), jnp.float32),
            jax.ShapeDtypeStruct((N, H), jnp.float32),
        ],
    )(x, fcwT, al, ar)


def _sc_gather(table, idx2d, width):
    R = idx2d.shape[0]
    RW = R // NW
    mesh = plsc.VectorSubcoreMesh(core_axis_name="c", subcore_axis_name="s")

    @functools.partial(
        pl.kernel,
        mesh=mesh,
        out_type=jax.ShapeDtypeStruct((R * 128, width), table.dtype),
        scratch_types=[
            pltpu.VMEM((128,), jnp.int32),
            pltpu.VMEM((128, width), table.dtype),
            pltpu.SemaphoreType.DMA,
        ],
    )
    def k(tab_hbm, idx_hbm, out_hbm, idx_v, rows_v, sem):
        wid = lax.axis_index("s") * 2 + lax.axis_index("c")

        @pl.loop(0, RW)
        def _(r):
            row = wid * RW + r
            pltpu.sync_copy(idx_hbm.at[row], idx_v)
            pltpu.async_copy(tab_hbm.at[idx_v], rows_v, sem).wait()
            pltpu.sync_copy(rows_v, out_hbm.at[pl.ds(row * 128, 128)])

    return k(table, idx2d)


def _sc_edge_gather(el_flat, er_flat, lab_f, src_pad, neg_pad):
    per_w = EP // NW
    CH = 1280
    mesh = plsc.VectorSubcoreMesh(core_axis_name="c", subcore_axis_name="s")
    cp = pltpu.CompilerParams()
    if "needs_layout_passes" in pltpu.CompilerParams.__dataclass_fields__:
        cp = dataclasses.replace(cp, needs_layout_passes=False)

    @functools.partial(
        pl.kernel,
        mesh=mesh,
        compiler_params=cp,
        out_type=jax.ShapeDtypeStruct((EP * 16,), jnp.float32),
        scratch_types=[
            pltpu.VMEM((N * H,), jnp.float32),
            pltpu.VMEM((N * H,), jnp.float32),
            pltpu.VMEM((N,), jnp.float32),
            pltpu.VMEM((CH,), jnp.int32),
            pltpu.VMEM((CH,), jnp.int32),
            pltpu.VMEM((CH * 16,), jnp.float32),
        ],
    )
    def k(el_hbm, er_hbm, lab_hbm, src_hbm, neg_hbm, out_hbm,
          el_v, er_v, lab_v, src_v, neg_v, out_v):
        pltpu.sync_copy(el_hbm, el_v)
        pltpu.sync_copy(er_hbm, er_v)
        pltpu.sync_copy(lab_hbm, lab_v)
        wid = lax.axis_index("s") * 2 + lax.axis_index("c")
        base = wid * per_w

        @pl.loop(0, per_w // CH)
        def _(ci):
            cb = base + ci * CH
            pltpu.sync_copy(src_hbm.at[pl.ds(cb, CH)], src_v)
            pltpu.sync_copy(neg_hbm.at[pl.ds(cb, CH)], neg_v)

            @pl.loop(0, CH // 16)
            def _(g):
                off = g * 16
                sidx = src_v[pl.ds(off, 16)]
                nidx = neg_v[pl.ds(off, 16)]
                s4 = sidx * 4
                n4 = nidx * 4
                rows16 = (lax.iota(jnp.int32, 16) + off) * 16
                for c in range(H):
                    v = plsc.load_gather(el_v, [s4 + c])
                    plsc.store_scatter(out_v, [rows16 + c], v)
                    w = plsc.load_gather(er_v, [n4 + c])
                    plsc.store_scatter(out_v, [rows16 + (4 + c)], w)
                lv = plsc.load_gather(lab_v, [sidx])
                plsc.store_scatter(out_v, [rows16 + 8], lv)

            pltpu.sync_copy(out_v, out_hbm.at[pl.ds(cb * 16, CH * 16)])

    return k(el_flat, er_flat, lab_f, src_pad, neg_pad)


def _stage_c_body(er_ref, lab_ref, g_ref, xg_ref, w_ref,
                  rst_ref, gl_ref, cl_ref):
    er_n = er_ref[...]
    lab_n = lab_ref[...]
    g = g_ref[...].reshape(BC, DEG, 16)
    el_s = g[:, :, 0:4]
    er_ng = g[:, :, 4:8]
    lab_s = g[:, :, 8]

    zp = el_s + er_n[:, None, :]
    e = jnp.where(zp >= 0, zp, SLOPE * zp)
    zn = el_s + er_ng
    en = jnp.where(zn >= 0, zn, SLOPE * zn)

    gl = jnp.maximum(en[:, None, :, :] + GM - e[:, :, None, :], 0.0)
    gl_sum = gl.sum()
    adj = lab_s == lab_n
    valid = adj[:, :, None] & (~adj)[:, None, :]
    ct = jnp.maximum(e[:, None, :, :] + CM - e[:, :, None, :], 0.0)
    cl_sum = (ct.sum(-1) * valid.astype(jnp.float32)).sum()

    gt = e[:, :, None, :] > e[:, None, :, :]
    eq = e[:, :, None, :] == e[:, None, :, :]
    dp = lax.broadcasted_iota(jnp.int32, (1, DEG, DEG, 1), 1)
    dd = lax.broadcasted_iota(jnp.int32, (1, DEG, DEG, 1), 2)
    rank = (gt | (eq & (dp < dd))).astype(jnp.float32).sum(1)
    keep = rank < K

    emax = e.max(axis=1, keepdims=True)
    num = jnp.where(keep, jnp.exp(e - emax), 0.0)
    den = num.sum(1, keepdims=True)
    a = num / den

    xgr = xg_ref[...].reshape(BC, DEG, DIN)
    xagg = (a[:, :, :, None] * xgr[:, :, None, :]).sum(1)
    outs = []
    for h in range(H):
        wh = w_ref[:, h * DOUT:(h + 1) * DOUT]
        outs.append(jnp.dot(xagg[:, h, :], wh,
                            preferred_element_type=jnp.float32,
                            precision=lax.Precision.HIGHEST))
    rst_ref[...] = jnp.concatenate(outs, axis=1)

    @pl.when(pl.program_id(0) == 0)
    def _():
        gl_ref[...] = jnp.zeros((1, 1), jnp.float32)
        cl_ref[...] = jnp.zeros((1, 1), jnp.float32)
    gl_ref[...] += (gl_sum / (N * H)).reshape(1, 1)
    cl_ref[...] += (cl_sum / (N * H)).reshape(1, 1)


def _stage_c(er, labf, g, xg, fcwT):
    return pl.pallas_call(
        _stage_c_body,
        grid=(N // BC,),
        in_specs=[
            pl.BlockSpec((BC, H), lambda i: (i, 0)),
            pl.BlockSpec((BC, 1), lambda i: (i, 0)),
            pl.BlockSpec((BC, DEG * 16), lambda i: (i, 0)),
            pl.BlockSpec((BC, DEG * DIN), lambda i: (i, 0)),
            pl.BlockSpec((DIN, H * DOUT), lambda i: (0, 0)),
        ],
        out_specs=[
            pl.BlockSpec((BC, H * DOUT), lambda i: (i, 0)),
            pl.BlockSpec((1, 1), lambda i: (0, 0)),
            pl.BlockSpec((1, 1), lambda i: (0, 0)),
        ],
        out_shape=[
            jax.ShapeDtypeStruct((N, H * DOUT), jnp.float32),
            jax.ShapeDtypeStruct((1, 1), jnp.float32),
            jax.ShapeDtypeStruct((1, 1), jnp.float32),
        ],
    )(er, labf, g, xg, fcwT)


def kernel(x, src, neg_dst, labels, fc_w, attn_l, attn_r):
    fcwT = fc_w.T
    al = attn_l.reshape(H, DOUT)
    ar = attn_r.reshape(H, DOUT)
    labf = labels.astype(jnp.float32)

    src_pad = jnp.pad(src.reshape(-1), (0, EP - E))
    neg_pad = jnp.pad(neg_dst.reshape(-1), (0, EP - E))
    xg = _sc_gather(x, src_pad.reshape(1280, 128), DIN)
    xg = xg.reshape(EP // DEG, DEG * DIN)

    el, er = _stage_a(x, fcwT, al, ar)

    g = _sc_edge_gather(el.reshape(-1), er.reshape(-1), labf,
                        src_pad, neg_pad)
    g = g.reshape(EP // DEG, DEG * 16)

    rst, gls, cls = _stage_c(er, labf.reshape(N, 1), g, xg, fcwT)
    return (rst.reshape(N, H, DOUT), gls[0, 0], cls[0, 0])


# speedup vs baseline: 7.7022x; 2.1692x over previous
import dataclasses
import functools

import jax
import jax.numpy as jnp
from jax import lax
from jax.experimental import pallas as pl
from jax.experimental.pallas import tpu as pltpu
from jax.experimental.pallas import tpu_sc as plsc

N, DEG, DIN, DOUT, H, K = 10000, 16, 128, 128, 4, 8
GM, CM, SLOPE = 0.1, 0.1, 0.2
E = N * DEG
EP = 1280 * 128
NW = 32

BA = 400
BC = 200


def _stage_a_body(x_ref, w_ref, al_ref, ar_ref, el_ref, er_ref):
    xb = x_ref[...]
    hb = jnp.dot(xb, w_ref[...], preferred_element_type=jnp.float32)
    h4 = hb.reshape(BA, H, DOUT)
    el_ref[...] = (h4 * al_ref[...][None]).sum(-1)
    er_ref[...] = (h4 * ar_ref[...][None]).sum(-1)


def _stage_a(x, fcwT, al, ar):
    return pl.pallas_call(
        _stage_a_body,
        grid=(N // BA,),
        in_specs=[
            pl.BlockSpec((BA, DIN), lambda i: (i, 0)),
            pl.BlockSpec((DIN, H * DOUT), lambda i: (0, 0)),
            pl.BlockSpec((H, DOUT), lambda i: (0, 0)),
            pl.BlockSpec((H, DOUT), lambda i: (0, 0)),
        ],
        out_specs=[
            pl.BlockSpec((BA, H), lambda i: (i, 0)),
            pl.BlockSpec((BA, H), lambda i: (i, 0)),
        ],
        out_shape=[
            jax.ShapeDtypeStruct((N, H), jnp.float32),
            jax.ShapeDtypeStruct((N, H), jnp.float32),
        ],
    )(x, fcwT, al, ar)


def _sc_gather(table, idx2d, width):
    R = idx2d.shape[0]
    RW = R // NW
    mesh = plsc.VectorSubcoreMesh(core_axis_name="c", subcore_axis_name="s")

    @functools.partial(
        pl.kernel,
        mesh=mesh,
        out_type=jax.ShapeDtypeStruct((R * 128, width), table.dtype),
        scratch_types=[
            pltpu.VMEM((128,), jnp.int32),
            pltpu.VMEM((128, width), table.dtype),
            pltpu.SemaphoreType.DMA,
        ],
    )
    def k(tab_hbm, idx_hbm, out_hbm, idx_v, rows_v, sem):
        wid = lax.axis_index("s") * 2 + lax.axis_index("c")

        @pl.loop(0, RW)
        def _(r):
            row = wid * RW + r
            pltpu.sync_copy(idx_hbm.at[row], idx_v)
            pltpu.async_copy(tab_hbm.at[idx_v], rows_v, sem).wait()
            pltpu.sync_copy(rows_v, out_hbm.at[pl.ds(row * 128, 128)])

    return k(table, idx2d)


def _sc_edge_tables(el_flat, er_flat, lab_f, src_pad, neg_pad):
    per_w = EP // NW
    CH = 1024
    mesh = plsc.VectorSubcoreMesh(core_axis_name="c", subcore_axis_name="s")
    cp = pltpu.CompilerParams()
    if "needs_layout_passes" in pltpu.CompilerParams.__dataclass_fields__:
        cp = dataclasses.replace(cp, needs_layout_passes=False)
    o4 = jax.ShapeDtypeStruct((EP * 4,), jnp.float32)

    @functools.partial(
        pl.kernel,
        mesh=mesh,
        compiler_params=cp,
        out_type=[o4, o4, o4, o4],
        scratch_types=[
            pltpu.VMEM((N * H,), jnp.float32),
            pltpu.VMEM((N * H,), jnp.float32),
            pltpu.VMEM((N,), jnp.float32),
            pltpu.VMEM((CH,), jnp.int32),
            pltpu.VMEM((CH,), jnp.int32),
            pltpu.VMEM((CH * 4,), jnp.float32),
            pltpu.VMEM((CH * 4,), jnp.float32),
            pltpu.VMEM((CH * 4,), jnp.float32),
            pltpu.VMEM((CH * 4,), jnp.float32),
        ],
    )
    def k(el_hbm, er_hbm, lab_hbm, src_hbm, neg_hbm,
          els_hbm, ern_hbm, erd_hbm, adj_hbm,
          el_v, er_v, lab_v, src_v, neg_v, els_v, ern_v, erd_v, adj_v):
        pltpu.sync_copy(el_hbm, el_v)
        pltpu.sync_copy(er_hbm, er_v)
        pltpu.sync_copy(lab_hbm, lab_v)
        wid = lax.axis_index("s") * 2 + lax.axis_index("c")
        base = wid * per_w
        zero16 = jnp.zeros((16,), jnp.int32)

        @pl.loop(0, per_w // CH)
        def _(ci):
            cb = base + ci * CH
            pltpu.sync_copy(src_hbm.at[pl.ds(cb, CH)], src_v)
            pltpu.sync_copy(neg_hbm.at[pl.ds(cb, CH)], neg_v)

            @pl.loop(0, CH // 16)
            def _(g):
                off = g * 16
                node = jnp.minimum(cb // 16 + g, N - 1)
                nb = g * 64
                sidx = src_v[pl.ds(off, 16)]
                nidx = neg_v[pl.ds(off, 16)]
                s4 = sidx * 4
                n4 = nidx * 4
                lv = plsc.load_gather(lab_v, [sidx])
                ln = plsc.load_gather(lab_v, [zero16 + node])
                adjf = jnp.where(lv == ln, 1.0, 0.0).astype(jnp.float32)
                for c in range(H):
                    els_v[pl.ds(nb + 16 * c, 16)] = (
                        plsc.load_gather(el_v, [s4 + c]))
                    ern_v[pl.ds(nb + 16 * c, 16)] = (
                        plsc.load_gather(er_v, [n4 + c]))
                    erd_v[pl.ds(nb + 16 * c, 16)] = (
                        plsc.load_gather(er_v, [zero16 + (node * 4 + c)]))
                    adj_v[pl.ds(nb + 16 * c, 16)] = adjf

            pltpu.sync_copy(els_v, els_hbm.at[pl.ds(cb * 4, CH * 4)])
            pltpu.sync_copy(ern_v, ern_hbm.at[pl.ds(cb * 4, CH * 4)])
            pltpu.sync_copy(erd_v, erd_hbm.at[pl.ds(cb * 4, CH * 4)])
            pltpu.sync_copy(adj_v, adj_hbm.at[pl.ds(cb * 4, CH * 4)])

    return k(el_flat, er_flat, lab_f, src_pad, neg_pad)


def _leaky(z):
    return jnp.where(z >= 0, z, SLOPE * z)


def _stage_c_body(els_ref, ern_ref, erd_ref, adj_ref, xg_ref, w_ref,
                  rst_ref, gl_ref, cl_ref):
    e = _leaky(els_ref[...] + erd_ref[...])
    en = _leaky(els_ref[...] + ern_ref[...])
    adj = adj_ref[...]
    dl = lax.broadcasted_iota(jnp.int32, (1, 64), 1) % 16

    def grp_roll(arr, s):
        s = s % 16
        if s == 0:
            return arr
        u = jnp.concatenate([arr[:, s:], arr[:, :s]], axis=1)
        v = jnp.concatenate([arr[:, 48 + s:], arr[:, :48 + s]], axis=1)
        return jnp.where(dl < 16 - s, u, v)

    gl_acc = jnp.maximum(en + GM - e, 0.0)
    cl_acc = jnp.zeros((BC, 64), jnp.float32)
    rank = jnp.zeros((BC, 64), jnp.float32)
    one = jnp.float32(1.0)
    for s in range(1, DEG):
        e_r = grp_roll(e, s)
        en_r = grp_roll(en, s)
        adj_r = grp_roll(adj, s)
        tie = (e_r == e) & (dl >= 16 - s)
        rank += jnp.where((e_r > e) | tie, one, 0.0)
        gl_acc += jnp.maximum(en_r + GM - e, 0.0)
        cl_acc += jnp.maximum(e_r + CM - e, 0.0) * (adj * (one - adj_r))
    gl_sum = gl_acc.sum()
    cl_sum = cl_acc.sum()

    m = e
    for k in (1, 2, 4, 8):
        m = jnp.maximum(m, grp_roll(m, k))
    num = jnp.where(rank < K, jnp.exp(e - m), 0.0)
    den = num
    for k in (1, 2, 4, 8):
        den = den + grp_roll(den, k)
    a = num / den

    xg = xg_ref[...]
    accs = [jnp.zeros((BC, DIN), jnp.float32) for _ in range(H)]
    for d in range(DEG):
        xd = xg[:, d * DIN:(d + 1) * DIN]
        for h in range(H):
            c = 16 * h + d
            accs[h] = accs[h] + a[:, c:c + 1] * xd
    outs = []
    for h in range(H):
        wh = w_ref[:, h * DOUT:(h + 1) * DOUT]
        outs.append(jnp.dot(accs[h], wh,
                            preferred_element_type=jnp.float32,
                            precision=lax.Precision.HIGHEST))
    rst_ref[...] = jnp.concatenate(outs, axis=1)

    @pl.when(pl.program_id(0) == 0)
    def _():
        gl_ref[...] = jnp.zeros((1, 1), jnp.float32)
        cl_ref[...] = jnp.zeros((1, 1), jnp.float32)
    gl_ref[...] += (gl_sum / (N * H)).reshape(1, 1)
    cl_ref[...] += (cl_sum / (N * H)).reshape(1, 1)


def _stage_c(els, ern, erd, adj, xg, fcwT):
    return pl.pallas_call(
        _stage_c_body,
        grid=(N // BC,),
        in_specs=[
            pl.BlockSpec((BC, 64), lambda i: (i, 0)),
            pl.BlockSpec((BC, 64), lambda i: (i, 0)),
            pl.BlockSpec((BC, 64), lambda i: (i, 0)),
            pl.BlockSpec((BC, 64), lambda i: (i, 0)),
            pl.BlockSpec((BC, DEG * DIN), lambda i: (i, 0)),
            pl.BlockSpec((DIN, H * DOUT), lambda i: (0, 0)),
        ],
        out_specs=[
            pl.BlockSpec((BC, H * DOUT), lambda i: (i, 0)),
            pl.BlockSpec((1, 1), lambda i: (0, 0)),
            pl.BlockSpec((1, 1), lambda i: (0, 0)),
        ],
        out_shape=[
            jax.ShapeDtypeStruct((N, H * DOUT), jnp.float32),
            jax.ShapeDtypeStruct((1, 1), jnp.float32),
            jax.ShapeDtypeStruct((1, 1), jnp.float32),
        ],
    )(els, ern, erd, adj, xg, fcwT)


def kernel(x, src, neg_dst, labels, fc_w, attn_l, attn_r):
    fcwT = fc_w.T
    al = attn_l.reshape(H, DOUT)
    ar = attn_r.reshape(H, DOUT)
    labf = labels.astype(jnp.float32)

    src_pad = jnp.pad(src.reshape(-1), (0, EP - E))
    neg_pad = jnp.pad(neg_dst.reshape(-1), (0, EP - E))
    xg = _sc_gather(x, src_pad.reshape(1280, 128), DIN)
    xg = xg.reshape(EP // DEG, DEG * DIN)

    el, er = _stage_a(x, fcwT, al, ar)

    els, ern, erd, adj = _sc_edge_tables(el.reshape(-1), er.reshape(-1),
                                         labf, src_pad, neg_pad)
    nrow = EP // DEG
    els = els.reshape(nrow, 64)
    ern = ern.reshape(nrow, 64)
    erd = erd.reshape(nrow, 64)
    adj = adj.reshape(nrow, 64)

    rst, gls, cls = _stage_c(els, ern, erd, adj, xg, fcwT)
    return (rst.reshape(N, H, DOUT), gls[0, 0], cls[0, 0])


# speedup vs baseline: 12.1245x; 1.5742x over previous
import dataclasses
import functools

import jax
import jax.numpy as jnp
from jax import lax
from jax.experimental import pallas as pl
from jax.experimental.pallas import tpu as pltpu
from jax.experimental.pallas import tpu_sc as plsc

N, DEG, DIN, DOUT, H, K = 10000, 16, 128, 128, 4, 8
GM, CM, SLOPE = 0.1, 0.1, 0.2
E = N * DEG
EP = 1280 * 128
NW = 32

BA = 400
BC = 200


def _stage_a_body(x_ref, w_ref, al_ref, ar_ref, el_ref, er_ref):
    xb = x_ref[...]
    hb = jnp.dot(xb, w_ref[...], preferred_element_type=jnp.float32)
    h4 = hb.reshape(BA, H, DOUT)
    el_ref[...] = (h4 * al_ref[...][None]).sum(-1)
    er_ref[...] = (h4 * ar_ref[...][None]).sum(-1)


def _stage_a(x, fcwT, al, ar):
    return pl.pallas_call(
        _stage_a_body,
        grid=(N // BA,),
        in_specs=[
            pl.BlockSpec((BA, DIN), lambda i: (i, 0)),
            pl.BlockSpec((DIN, H * DOUT), lambda i: (0, 0)),
            pl.BlockSpec((H, DOUT), lambda i: (0, 0)),
            pl.BlockSpec((H, DOUT), lambda i: (0, 0)),
        ],
        out_specs=[
            pl.BlockSpec((BA, H), lambda i: (i, 0)),
            pl.BlockSpec((BA, H), lambda i: (i, 0)),
        ],
        out_shape=[
            jax.ShapeDtypeStruct((N, H), jnp.float32),
            jax.ShapeDtypeStruct((N, H), jnp.float32),
        ],
    )(x, fcwT, al, ar)


def _sc_gather(table, idx2d, width):
    R = idx2d.shape[0]
    RW = R // NW
    mesh = plsc.VectorSubcoreMesh(core_axis_name="c", subcore_axis_name="s")

    @functools.partial(
        pl.kernel,
        mesh=mesh,
        out_type=jax.ShapeDtypeStruct((R * 128, width), table.dtype),
        scratch_types=[
            pltpu.VMEM((128,), jnp.int32),
            pltpu.VMEM((128, width), table.dtype),
            pltpu.SemaphoreType.DMA,
        ],
    )
    def k(tab_hbm, idx_hbm, out_hbm, idx_v, rows_v, sem):
        wid = lax.axis_index("s") * 2 + lax.axis_index("c")

        @pl.loop(0, RW)
        def _(r):
            row = wid * RW + r
            pltpu.sync_copy(idx_hbm.at[row], idx_v)
            pltpu.async_copy(tab_hbm.at[idx_v], rows_v, sem).wait()
            pltpu.sync_copy(rows_v, out_hbm.at[pl.ds(row * 128, 128)])

    return k(table, idx2d)


def _sc_edge_tables(el_flat, er_flat, lab_f, src_pad, neg_pad):
    per_w = EP // NW
    CH = 1024
    mesh = plsc.VectorSubcoreMesh(core_axis_name="c", subcore_axis_name="s")
    cp = pltpu.CompilerParams()
    if "needs_layout_passes" in pltpu.CompilerParams.__dataclass_fields__:
        cp = dataclasses.replace(cp, needs_layout_passes=False)
    o4 = jax.ShapeDtypeStruct((EP * 4,), jnp.float32)

    @functools.partial(
        pl.kernel,
        mesh=mesh,
        compiler_params=cp,
        out_type=[o4, o4, o4, o4],
        scratch_types=[
            pltpu.VMEM((N * H,), jnp.float32),
            pltpu.VMEM((N * H,), jnp.float32),
            pltpu.VMEM((N,), jnp.float32),
            pltpu.VMEM((CH,), jnp.int32),
            pltpu.VMEM((CH,), jnp.int32),
            pltpu.VMEM((CH * 4,), jnp.float32),
            pltpu.VMEM((CH * 4,), jnp.float32),
            pltpu.VMEM((CH * 4,), jnp.float32),
            pltpu.VMEM((CH * 4,), jnp.float32),
        ],
    )
    def k(el_hbm, er_hbm, lab_hbm, src_hbm, neg_hbm,
          els_hbm, ern_hbm, erd_hbm, adj_hbm,
          el_v, er_v, lab_v, src_v, neg_v, els_v, ern_v, erd_v, adj_v):
        pltpu.sync_copy(el_hbm, el_v)
        pltpu.sync_copy(er_hbm, er_v)
        pltpu.sync_copy(lab_hbm, lab_v)
        wid = lax.axis_index("s") * 2 + lax.axis_index("c")
        base = wid * per_w
        zero16 = jnp.zeros((16,), jnp.int32)

        @pl.loop(0, per_w // CH)
        def _(ci):
            cb = base + ci * CH
            pltpu.sync_copy(src_hbm.at[pl.ds(cb, CH)], src_v)
            pltpu.sync_copy(neg_hbm.at[pl.ds(cb, CH)], neg_v)

            @pl.loop(0, CH // 16)
            def _(g):
                off = g * 16
                node = jnp.minimum(cb // 16 + g, N - 1)
                nb = g * 64
                sidx = src_v[pl.ds(off, 16)]
                nidx = neg_v[pl.ds(off, 16)]
                s4 = sidx * 4
                n4 = nidx * 4
                lv = plsc.load_gather(lab_v, [sidx])
                ln = plsc.load_gather(lab_v, [zero16 + node])
                adjf = jnp.where(lv == ln, 1.0, 0.0).astype(jnp.float32)
                for c in range(H):
                    els_v[pl.ds(nb + 16 * c, 16)] = (
                        plsc.load_gather(el_v, [s4 + c]))
                    ern_v[pl.ds(nb + 16 * c, 16)] = (
                        plsc.load_gather(er_v, [n4 + c]))
                    erd_v[pl.ds(nb + 16 * c, 16)] = (
                        plsc.load_gather(er_v, [zero16 + (node * 4 + c)]))
                    adj_v[pl.ds(nb + 16 * c, 16)] = adjf

            pltpu.sync_copy(els_v, els_hbm.at[pl.ds(cb * 4, CH * 4)])
            pltpu.sync_copy(ern_v, ern_hbm.at[pl.ds(cb * 4, CH * 4)])
            pltpu.sync_copy(erd_v, erd_hbm.at[pl.ds(cb * 4, CH * 4)])
            pltpu.sync_copy(adj_v, adj_hbm.at[pl.ds(cb * 4, CH * 4)])

    return k(el_flat, er_flat, lab_f, src_pad, neg_pad)


def _leaky(z):
    return jnp.where(z >= 0, z, SLOPE * z)


def _stage_c_body(els_ref, ern_ref, erd_ref, adj_ref, xg_ref, w_ref,
                  rst_ref, gl_ref, cl_ref):
    e = _leaky(els_ref[...] + erd_ref[...])
    en = _leaky(els_ref[...] + ern_ref[...])
    adj = adj_ref[...]
    dl = lax.broadcasted_iota(jnp.int32, (1, 128), 1) % 16

    def grp_roll(arr, s):
        s = s % 16
        if s == 0:
            return arr
        u = jnp.concatenate([arr[:, s:], arr[:, :s]], axis=1)
        w = (s + 112) % 128
        v = jnp.concatenate([arr[:, w:], arr[:, :w]], axis=1)
        return jnp.where(dl < 16 - s, u, v)

    gl_acc = jnp.maximum(en + GM - e, 0.0)
    cl_acc = jnp.zeros((BC, 128), jnp.float32)
    rank = jnp.zeros((BC, 128), jnp.float32)
    one = jnp.float32(1.0)
    nadj = one - adj
    for s in range(1, DEG):
        e_r = grp_roll(e, s)
        adj_r = grp_roll(adj, s)
        tie = (e_r == e) & (dl >= 16 - s)
        rank += jnp.where((e_r > e) | tie, one, 0.0)
        gl_acc += jnp.maximum(en + GM - e_r, 0.0)
        cl_acc += jnp.maximum(e + CM - e_r, 0.0) * (adj_r * nadj)
    gl_sum = gl_acc.sum()
    cl_sum = cl_acc.sum()

    m = e
    for k in (1, 2, 4, 8):
        m = jnp.maximum(m, grp_roll(m, k))
    num = jnp.where(rank < K, jnp.exp(e - m), 0.0)
    den = num
    for k in (1, 2, 4, 8):
        den = den + grp_roll(den, k)
    a = num / den

    xg = xg_ref[...]
    outs = []
    for p in range(2):
        for h in range(H):
            acc = jnp.zeros((BC, DIN), jnp.float32)
            for d in range(DEG):
                c = 64 * p + 16 * h + d
                xd = xg[:, (DEG * p + d) * DIN:(DEG * p + d + 1) * DIN]
                acc = acc + a[:, c:c + 1] * xd
            wh = w_ref[:, h * DOUT:(h + 1) * DOUT]
            outs.append(jnp.dot(acc, wh,
                                preferred_element_type=jnp.float32,
                                precision=lax.Precision.HIGHEST))
    rst_ref[...] = jnp.concatenate(outs, axis=1)

    @pl.when(pl.program_id(0) == 0)
    def _():
        gl_ref[...] = jnp.zeros((1, 1), jnp.float32)
        cl_ref[...] = jnp.zeros((1, 1), jnp.float32)
    gl_ref[...] += (gl_sum / (N * H)).reshape(1, 1)
    cl_ref[...] += (cl_sum / (N * H)).reshape(1, 1)


def _stage_c(els, ern, erd, adj, xg, fcwT):
    nrow = N // 2
    return pl.pallas_call(
        _stage_c_body,
        grid=(nrow // BC,),
        in_specs=[
            pl.BlockSpec((BC, 128), lambda i: (i, 0)),
            pl.BlockSpec((BC, 128), lambda i: (i, 0)),
            pl.BlockSpec((BC, 128), lambda i: (i, 0)),
            pl.BlockSpec((BC, 128), lambda i: (i, 0)),
            pl.BlockSpec((BC, 2 * DEG * DIN), lambda i: (i, 0)),
            pl.BlockSpec((DIN, H * DOUT), lambda i: (0, 0)),
        ],
        out_specs=[
            pl.BlockSpec((BC, 2 * H * DOUT), lambda i: (i, 0)),
            pl.BlockSpec((1, 1), lambda i: (0, 0)),
            pl.BlockSpec((1, 1), lambda i: (0, 0)),
        ],
        out_shape=[
            jax.ShapeDtypeStruct((nrow, 2 * H * DOUT), jnp.float32),
            jax.ShapeDtypeStruct((1, 1), jnp.float32),
            jax.ShapeDtypeStruct((1, 1), jnp.float32),
        ],
    )(els, ern, erd, adj, xg, fcwT)


def kernel(x, src, neg_dst, labels, fc_w, attn_l, attn_r):
    fcwT = fc_w.T
    al = attn_l.reshape(H, DOUT)
    ar = attn_r.reshape(H, DOUT)
    labf = labels.astype(jnp.float32)

    src_pad = jnp.pad(src.reshape(-1), (0, EP - E))
    neg_pad = jnp.pad(neg_dst.reshape(-1), (0, EP - E))
    xg = _sc_gather(x, src_pad.reshape(1280, 128), DIN)
    xg = xg.reshape(EP // 32, 2 * DEG * DIN)

    el, er = _stage_a(x, fcwT, al, ar)

    els, ern, erd, adj = _sc_edge_tables(el.reshape(-1), er.reshape(-1),
                                         labf, src_pad, neg_pad)
    nrow = EP // 32
    els = els.reshape(nrow, 128)
    ern = ern.reshape(nrow, 128)
    erd = erd.reshape(nrow, 128)
    adj = adj.reshape(nrow, 128)

    rst, gls, cls = _stage_c(els, ern, erd, adj, xg, fcwT)
    return (rst.reshape(N, H, DOUT), gls[0, 0], cls[0, 0])


# speedup vs baseline: 13.2209x; 1.0904x over previous
import dataclasses
import functools

import jax
import jax.numpy as jnp
from jax import lax
from jax.experimental import pallas as pl
from jax.experimental.pallas import tpu as pltpu
from jax.experimental.pallas import tpu_sc as plsc

N, DEG, DIN, DOUT, H, K = 10000, 16, 128, 128, 4, 8
GM, CM, SLOPE = 0.1, 0.1, 0.2
E = N * DEG
EP = 1280 * 128
NW = 32

BA = 400
BC = 200


def _stage_a_body(x_ref, w_ref, al_ref, ar_ref, el_ref, er_ref):
    xb = x_ref[...]
    hb = jnp.dot(xb, w_ref[...], preferred_element_type=jnp.float32)
    h4 = hb.reshape(BA, H, DOUT)
    el_ref[...] = (h4 * al_ref[...][None]).sum(-1)
    er_ref[...] = (h4 * ar_ref[...][None]).sum(-1)


def _stage_a(x, fcwT, al, ar):
    return pl.pallas_call(
        _stage_a_body,
        grid=(N // BA,),
        in_specs=[
            pl.BlockSpec((BA, DIN), lambda i: (i, 0)),
            pl.BlockSpec((DIN, H * DOUT), lambda i: (0, 0)),
            pl.BlockSpec((H, DOUT), lambda i: (0, 0)),
            pl.BlockSpec((H, DOUT), lambda i: (0, 0)),
        ],
        out_specs=[
            pl.BlockSpec((BA, H), lambda i: (i, 0)),
            pl.BlockSpec((BA, H), lambda i: (i, 0)),
        ],
        out_shape=[
            jax.ShapeDtypeStruct((N, H), jnp.float32),
            jax.ShapeDtypeStruct((N, H), jnp.float32),
        ],
    )(x, fcwT, al, ar)


def _sc_gather(table, idx_flat, width):
    R = idx_flat.shape[0] // 128
    RW = R // NW
    mesh = plsc.VectorSubcoreMesh(core_axis_name="c", subcore_axis_name="s")

    @functools.partial(
        pl.kernel,
        mesh=mesh,
        out_type=jax.ShapeDtypeStruct((R * 128, width), table.dtype),
        scratch_types=[
            pltpu.VMEM((RW * 128,), jnp.int32),
            pltpu.VMEM((128, width), table.dtype),
            pltpu.VMEM((128, width), table.dtype),
            pltpu.SemaphoreType.DMA,
            pltpu.SemaphoreType.DMA,
            pltpu.SemaphoreType.DMA,
            pltpu.SemaphoreType.DMA,
        ],
    )
    def k(tab_hbm, idx_hbm, out_hbm, idx_v, ra, rb, gsa, gsb, wsa, wsb):
        wid = lax.axis_index("s") * 2 + lax.axis_index("c")
        base = wid * RW
        pltpu.sync_copy(idx_hbm.at[pl.ds(base * 128, RW * 128)], idx_v)
        bufs, gsems, wsems = [ra, rb], [gsa, gsb], [wsa, wsb]
        gcop = [None, None]
        wcop = [None, None]

        def gather(r, b):
            src = tab_hbm.at[idx_v.at[pl.ds(r * 128, 128)]]
            gcop[b] = pltpu.async_copy(src, bufs[b], gsems[b])

        gather(0, 0)
        for r in range(RW):
            b = r & 1
            nb = 1 - b
            if r + 1 < RW:
                if wcop[nb] is not None:
                    wcop[nb].wait()
                gather(r + 1, nb)
            gcop[b].wait()
            wcop[b] = pltpu.async_copy(
                bufs[b], out_hbm.at[pl.ds((base + r) * 128, 128)], wsems[b])
        wcop[0].wait()
        wcop[1].wait()

    return k(table, idx_flat)


def _sc_edge_tables(el_flat, er_flat, lab_f, src_pad, neg_pad):
    per_w = EP // NW
    CH = 1024
    mesh = plsc.VectorSubcoreMesh(core_axis_name="c", subcore_axis_name="s")
    cp = pltpu.CompilerParams()
    if "needs_layout_passes" in pltpu.CompilerParams.__dataclass_fields__:
        cp = dataclasses.replace(cp, needs_layout_passes=False)
    o4 = jax.ShapeDtypeStruct((EP * 4,), jnp.float32)

    @functools.partial(
        pl.kernel,
        mesh=mesh,
        compiler_params=cp,
        out_type=[o4, o4, o4, o4],
        scratch_types=[
            pltpu.VMEM((N * H,), jnp.float32),
            pltpu.VMEM((N * H,), jnp.float32),
            pltpu.VMEM((N,), jnp.float32),
            pltpu.VMEM((CH,), jnp.int32),
            pltpu.VMEM((CH,), jnp.int32),
            pltpu.VMEM((CH * 4,), jnp.float32),
            pltpu.VMEM((CH * 4,), jnp.float32),
            pltpu.VMEM((CH * 4,), jnp.float32),
            pltpu.VMEM((CH * 4,), jnp.float32),
        ],
    )
    def k(el_hbm, er_hbm, lab_hbm, src_hbm, neg_hbm,
          els_hbm, ern_hbm, erd_hbm, adj_hbm,
          el_v, er_v, lab_v, src_v, neg_v, els_v, ern_v, erd_v, adj_v):
        pltpu.sync_copy(el_hbm, el_v)
        pltpu.sync_copy(er_hbm, er_v)
        pltpu.sync_copy(lab_hbm, lab_v)
        wid = lax.axis_index("s") * 2 + lax.axis_index("c")
        base = wid * per_w
        zero16 = jnp.zeros((16,), jnp.int32)

        @pl.loop(0, per_w // CH)
        def _(ci):
            cb = base + ci * CH
            pltpu.sync_copy(src_hbm.at[pl.ds(cb, CH)], src_v)
            pltpu.sync_copy(neg_hbm.at[pl.ds(cb, CH)], neg_v)

            @pl.loop(0, CH // 16)
            def _(g):
                off = g * 16
                node = jnp.minimum(cb // 16 + g, N - 1)
                nb = g * 64
                sidx = src_v[pl.ds(off, 16)]
                nidx = neg_v[pl.ds(off, 16)]
                s4 = sidx * 4
                n4 = nidx * 4
                lv = plsc.load_gather(lab_v, [sidx])
                ln = plsc.load_gather(lab_v, [zero16 + node])
                adjf = jnp.where(lv == ln, 1.0, 0.0).astype(jnp.float32)
                for c in range(H):
                    els_v[pl.ds(nb + 16 * c, 16)] = (
                        plsc.load_gather(el_v, [s4 + c]))
                    ern_v[pl.ds(nb + 16 * c, 16)] = (
                        plsc.load_gather(er_v, [n4 + c]))
                    erd_v[pl.ds(nb + 16 * c, 16)] = (
                        plsc.load_gather(er_v, [zero16 + (node * 4 + c)]))
                    adj_v[pl.ds(nb + 16 * c, 16)] = adjf

            pltpu.sync_copy(els_v, els_hbm.at[pl.ds(cb * 4, CH * 4)])
            pltpu.sync_copy(ern_v, ern_hbm.at[pl.ds(cb * 4, CH * 4)])
            pltpu.sync_copy(erd_v, erd_hbm.at[pl.ds(cb * 4, CH * 4)])
            pltpu.sync_copy(adj_v, adj_hbm.at[pl.ds(cb * 4, CH * 4)])

    return k(el_flat, er_flat, lab_f, src_pad, neg_pad)


def _leaky(z):
    return jnp.where(z >= 0, z, SLOPE * z)


def _stage_c_body(els_ref, ern_ref, erd_ref, adj_ref, xg_ref, w_ref,
                  rst_ref, gl_ref, cl_ref):
    e = _leaky(els_ref[...] + erd_ref[...])
    en = _leaky(els_ref[...] + ern_ref[...])
    adj = adj_ref[...]
    dl = lax.broadcasted_iota(jnp.int32, (1, 128), 1) % 16

    def grp_roll(arr, s):
        s = s % 16
        if s == 0:
            return arr
        u = jnp.concatenate([arr[:, s:], arr[:, :s]], axis=1)
        w = (s + 112) % 128
        v = jnp.concatenate([arr[:, w:], arr[:, :w]], axis=1)
        return jnp.where(dl < 16 - s, u, v)

    gl_acc = jnp.maximum(en + GM - e, 0.0)
    cl_acc = jnp.zeros((BC, 128), jnp.float32)
    rank = jnp.zeros((BC, 128), jnp.float32)
    one = jnp.float32(1.0)
    nadj = one - adj
    for s in range(1, DEG):
        e_r = grp_roll(e, s)
        adj_r = grp_roll(adj, s)
        tie = (e_r == e) & (dl >= 16 - s)
        rank += jnp.where((e_r > e) | tie, one, 0.0)
        gl_acc += jnp.maximum(en + GM - e_r, 0.0)
        cl_acc += jnp.maximum(e + CM - e_r, 0.0) * (adj_r * nadj)
    gl_sum = gl_acc.sum()
    cl_sum = cl_acc.sum()

    m = e
    for k in (1, 2, 4, 8):
        m = jnp.maximum(m, grp_roll(m, k))
    num = jnp.where(rank < K, jnp.exp(e - m), 0.0)
    den = num
    for k in (1, 2, 4, 8):
        den = den + grp_roll(den, k)
    a = num / den

    xg = xg_ref[...]
    outs = []
    for p in range(2):
        for h in range(H):
            acc = jnp.zeros((BC, DIN), jnp.float32)
            for d in range(DEG):
                c = 64 * p + 16 * h + d
                xd = xg[:, (DEG * p + d) * DIN:(DEG * p + d + 1) * DIN]
                acc = acc + a[:, c:c + 1] * xd
            wh = w_ref[:, h * DOUT:(h + 1) * DOUT]
            outs.append(jnp.dot(acc, wh,
                                preferred_element_type=jnp.float32,
                                precision=lax.Precision.HIGHEST))
    rst_ref[...] = jnp.concatenate(outs, axis=1)

    @pl.when(pl.program_id(0) == 0)
    def _():
        gl_ref[...] = jnp.zeros((1, 1), jnp.float32)
        cl_ref[...] = jnp.zeros((1, 1), jnp.float32)
    gl_ref[...] += (gl_sum / (N * H)).reshape(1, 1)
    cl_ref[...] += (cl_sum / (N * H)).reshape(1, 1)


def _stage_c(els, ern, erd, adj, xg, fcwT):
    nrow = N // 2
    return pl.pallas_call(
        _stage_c_body,
        grid=(nrow // BC,),
        in_specs=[
            pl.BlockSpec((BC, 128), lambda i: (i, 0)),
            pl.BlockSpec((BC, 128), lambda i: (i, 0)),
            pl.BlockSpec((BC, 128), lambda i: (i, 0)),
            pl.BlockSpec((BC, 128), lambda i: (i, 0)),
            pl.BlockSpec((BC, 2 * DEG * DIN), lambda i: (i, 0)),
            pl.BlockSpec((DIN, H * DOUT), lambda i: (0, 0)),
        ],
        out_specs=[
            pl.BlockSpec((BC, 2 * H * DOUT), lambda i: (i, 0)),
            pl.BlockSpec((1, 1), lambda i: (0, 0)),
            pl.BlockSpec((1, 1), lambda i: (0, 0)),
        ],
        out_shape=[
            jax.ShapeDtypeStruct((nrow, 2 * H * DOUT), jnp.float32),
            jax.ShapeDtypeStruct((1, 1), jnp.float32),
            jax.ShapeDtypeStruct((1, 1), jnp.float32),
        ],
    )(els, ern, erd, adj, xg, fcwT)


def kernel(x, src, neg_dst, labels, fc_w, attn_l, attn_r):
    fcwT = fc_w.T
    al = attn_l.reshape(H, DOUT)
    ar = attn_r.reshape(H, DOUT)
    labf = labels.astype(jnp.float32)

    src_pad = jnp.pad(src.reshape(-1), (0, EP - E))
    neg_pad = jnp.pad(neg_dst.reshape(-1), (0, EP - E))
    xg = _sc_gather(x, src_pad, DIN)
    xg = xg.reshape(EP // 32, 2 * DEG * DIN)

    el, er = _stage_a(x, fcwT, al, ar)

    els, ern, erd, adj = _sc_edge_tables(el.reshape(-1), er.reshape(-1),
                                         labf, src_pad, neg_pad)
    nrow = EP // 32
    els = els.reshape(nrow, 128)
    ern = ern.reshape(nrow, 128)
    erd = erd.reshape(nrow, 128)
    adj = adj.reshape(nrow, 128)

    rst, gls, cls = _stage_c(els, ern, erd, adj, xg, fcwT)
    return (rst.reshape(N, H, DOUT), gls[0, 0], cls[0, 0])
